# baseline (speedup 1.0000x reference)
import functools

import jax
import jax.numpy as jnp
from jax import lax
from jax.experimental import pallas as pl
from jax.experimental.pallas import tpu as pltpu
from jax.experimental.pallas import tpu_sc as plsc

BATCH = 16384
EMBED = 32
HIDDEN = 64
BLK = 2048


def _sc_gather(user_id, item_id, user_table, item_table):
    info = plsc.get_sparse_core_info()
    nc, ns = info.num_cores, info.num_subcores
    nw = nc * ns
    b_per_w = BATCH // nw
    mesh = plsc.VectorSubcoreMesh(core_axis_name="c", subcore_axis_name="s")

    @functools.partial(
        pl.kernel,
        mesh=mesh,
        compiler_params=pltpu.CompilerParams(use_tc_tiling_on_sc=False),
        out_type=(
            jax.ShapeDtypeStruct((BATCH, EMBED), jnp.float32),
            jax.ShapeDtypeStruct((BATCH, EMBED), jnp.float32),
        ),
        scratch_types=[
            pltpu.VMEM((b_per_w,), jnp.int32),
            pltpu.VMEM((b_per_w,), jnp.int32),
            pltpu.VMEM((b_per_w, EMBED), jnp.float32),
            pltpu.VMEM((b_per_w, EMBED), jnp.float32),
            pltpu.SemaphoreType.DMA,
            pltpu.SemaphoreType.DMA,
        ],
    )
    def gather_k(uid_hbm, iid_hbm, ut_hbm, it_hbm, ue_hbm, ie_hbm,
                 uidx_v, iidx_v, urows_v, irows_v, sem_u, sem_i):
        wid = lax.axis_index("s") * nc + lax.axis_index("c")
        base = wid * b_per_w
        pltpu.sync_copy(uid_hbm.at[pl.ds(base, b_per_w)], uidx_v)
        pltpu.sync_copy(iid_hbm.at[pl.ds(base, b_per_w)], iidx_v)
        cu = pltpu.async_copy(ut_hbm.at[uidx_v], urows_v, sem_u)
        ci = pltpu.async_copy(it_hbm.at[iidx_v], irows_v, sem_i)
        cu.wait()
        ci.wait()
        pltpu.sync_copy(urows_v, ue_hbm.at[pl.ds(base, b_per_w)])
        pltpu.sync_copy(irows_v, ie_hbm.at[pl.ds(base, b_per_w)])

    return gather_k(user_id, item_id, user_table, item_table)


def _mlp_body(ue_ref, ie_ref, w1u_ref, w1v_ref, b1_ref, w2t_ref, b2_ref, out_ref):
    h = jnp.dot(ue_ref[...], w1u_ref[...], preferred_element_type=jnp.float32)
    h = h + jnp.dot(ie_ref[...], w1v_ref[...], preferred_element_type=jnp.float32)
    h = jnp.maximum(h + b1_ref[...], 0.0)
    out_ref[...] = jnp.dot(h, w2t_ref[...], preferred_element_type=jnp.float32) + b2_ref[0, 0]


def _tc_mlp(ue, ie, w1u, w1v, b1_2d, w2t, b2_2d):
    grid = (BATCH // BLK,)
    return pl.pallas_call(
        _mlp_body,
        grid=grid,
        in_specs=[
            pl.BlockSpec((BLK, EMBED), lambda i: (i, 0)),
            pl.BlockSpec((BLK, EMBED), lambda i: (i, 0)),
            pl.BlockSpec((EMBED, HIDDEN), lambda i: (0, 0)),
            pl.BlockSpec((EMBED, HIDDEN), lambda i: (0, 0)),
            pl.BlockSpec((1, HIDDEN), lambda i: (0, 0)),
            pl.BlockSpec((HIDDEN, 1), lambda i: (0, 0)),
            pl.BlockSpec((1, 1), lambda i: (0, 0)),
        ],
        out_specs=pl.BlockSpec((BLK, 1), lambda i: (i, 0)),
        out_shape=jax.ShapeDtypeStruct((BATCH, 1), jnp.float32),
    )(ue, ie, w1u, w1v, b1_2d, w2t, b2_2d)


def kernel(user_id, item_id, user_table, item_table, W1, b1, W2, b2):
    uid = user_id.astype(jnp.int32)
    iid = item_id.astype(jnp.int32)
    ue, ie = _sc_gather(uid, iid, user_table, item_table)
    w1u = W1[:, :EMBED].T
    w1v = W1[:, EMBED:].T
    out = _tc_mlp(ue, ie, w1u, w1v, b1.reshape(1, HIDDEN), W2.T,
                  b2.reshape(1, 1))
    return out.reshape(BATCH)


# speedup vs baseline: 1.4886x; 1.4886x over previous
import functools

import jax
import jax.numpy as jnp
from jax import lax
from jax.experimental import pallas as pl
from jax.experimental.pallas import tpu as pltpu
from jax.experimental.pallas import tpu_sc as plsc

BATCH = 16384
EMBED = 32
HIDDEN = 64
BLK = 2048
CHUNK = 128


def _sc_gather(user_id, item_id, user_table, item_table):
    info = plsc.get_sparse_core_info()
    nc, ns = info.num_cores, info.num_subcores
    nw = nc * ns
    b_per_w = BATCH // nw
    mesh = plsc.VectorSubcoreMesh(core_axis_name="c", subcore_axis_name="s")

    @functools.partial(
        pl.kernel,
        mesh=mesh,
        out_type=(
            jax.ShapeDtypeStruct((BATCH, EMBED), jnp.float32),
            jax.ShapeDtypeStruct((BATCH, EMBED), jnp.float32),
        ),
        scratch_types=[
            pltpu.VMEM((b_per_w,), jnp.int32),
            pltpu.VMEM((b_per_w,), jnp.int32),
            pltpu.VMEM((CHUNK, EMBED), jnp.float32),
            pltpu.VMEM((CHUNK, EMBED), jnp.float32),
            pltpu.SemaphoreType.DMA,
            pltpu.SemaphoreType.DMA,
        ],
    )
    def gather_k(uid_hbm, iid_hbm, ut_hbm, it_hbm, ue_hbm, ie_hbm,
                 uidx_v, iidx_v, urows_v, irows_v, sem_u, sem_i):
        wid = lax.axis_index("s") * nc + lax.axis_index("c")
        base = wid * b_per_w
        pltpu.sync_copy(uid_hbm.at[pl.ds(base, b_per_w)], uidx_v)
        pltpu.sync_copy(iid_hbm.at[pl.ds(base, b_per_w)], iidx_v)

        @pl.loop(0, b_per_w // CHUNK)
        def _(c):
            c0 = c * CHUNK

            @pl.loop(0, CHUNK // 16)
            def _(g):
                j0 = g * 16
                uvec = uidx_v[pl.ds(c0 + j0, 16)]
                ivec = iidx_v[pl.ds(c0 + j0, 16)]
                for l in range(16):
                    pltpu.async_copy(ut_hbm.at[uvec[l]], urows_v.at[j0 + l],
                                     sem_u)
                    pltpu.async_copy(it_hbm.at[ivec[l]], irows_v.at[j0 + l],
                                     sem_i)

            @pl.loop(0, CHUNK)
            def _(j):
                pltpu.make_async_copy(ut_hbm.at[0], urows_v.at[0], sem_u).wait()
                pltpu.make_async_copy(it_hbm.at[0], irows_v.at[0], sem_i).wait()

            pltpu.sync_copy(urows_v, ue_hbm.at[pl.ds(base + c0, CHUNK)])
            pltpu.sync_copy(irows_v, ie_hbm.at[pl.ds(base + c0, CHUNK)])

    return gather_k(user_id, item_id, user_table, item_table)


def _mlp_body(ue_ref, ie_ref, w1u_ref, w1v_ref, b1_ref, w2t_ref, b2_ref, out_ref):
    h = jnp.dot(ue_ref[...], w1u_ref[...], preferred_element_type=jnp.float32)
    h = h + jnp.dot(ie_ref[...], w1v_ref[...], preferred_element_type=jnp.float32)
    h = jnp.maximum(h + b1_ref[...], 0.0)
    out_ref[...] = jnp.dot(h, w2t_ref[...], preferred_element_type=jnp.float32) + b2_ref[0, 0]


def _tc_mlp(ue, ie, w1u, w1v, b1_2d, w2t, b2_2d):
    grid = (BATCH // BLK,)
    return pl.pallas_call(
        _mlp_body,
        grid=grid,
        in_specs=[
            pl.BlockSpec((BLK, EMBED), lambda i: (i, 0)),
            pl.BlockSpec((BLK, EMBED), lambda i: (i, 0)),
            pl.BlockSpec((EMBED, HIDDEN), lambda i: (0, 0)),
            pl.BlockSpec((EMBED, HIDDEN), lambda i: (0, 0)),
            pl.BlockSpec((1, HIDDEN), lambda i: (0, 0)),
            pl.BlockSpec((HIDDEN, 1), lambda i: (0, 0)),
            pl.BlockSpec((1, 1), lambda i: (0, 0)),
        ],
        out_specs=pl.BlockSpec((BLK, 1), lambda i: (i, 0)),
        out_shape=jax.ShapeDtypeStruct((BATCH, 1), jnp.float32),
    )(ue, ie, w1u, w1v, b1_2d, w2t, b2_2d)


def kernel(user_id, item_id, user_table, item_table, W1, b1, W2, b2):
    uid = user_id.astype(jnp.int32)
    iid = item_id.astype(jnp.int32)
    ue, ie = _sc_gather(uid, iid, user_table, item_table)
    w1u = W1[:, :EMBED].T
    w1v = W1[:, EMBED:].T
    out = _tc_mlp(ue, ie, w1u, w1v, b1.reshape(1, HIDDEN), W2.T,
                  b2.reshape(1, 1))
    return out.reshape(BATCH)


# speedup vs baseline: 3.4187x; 2.2965x over previous
import functools

import jax
import jax.numpy as jnp
from jax import lax
from jax.experimental import pallas as pl
from jax.experimental.pallas import tpu as pltpu
from jax.experimental.pallas import tpu_sc as plsc

BATCH = 16384
EMBED = 32
HIDDEN = 64
BLK = 2048
LANES = 128
NBUF = 4


def _sc_gather(user_id, item_id, ut_t, it_t):
    info = plsc.get_sparse_core_info()
    nc, ns = info.num_cores, info.num_subcores
    nw = nc * ns
    b_per_w = BATCH // nw
    g16 = b_per_w // 16
    mesh = plsc.VectorSubcoreMesh(core_axis_name="c", subcore_axis_name="s")

    @functools.partial(
        pl.kernel,
        mesh=mesh,
        compiler_params=pltpu.CompilerParams(
            disable_bounds_checks=True, needs_layout_passes=False),
        out_type=jax.ShapeDtypeStruct((2 * EMBED * BATCH,), jnp.float32),
        scratch_types=[
            pltpu.VMEM((b_per_w,), jnp.int32),
            pltpu.VMEM((b_per_w,), jnp.int32),
            pltpu.VMEM((2 * EMBED * b_per_w,), jnp.float32),
        ]
        + [pltpu.VMEM((EMBED, LANES), jnp.float32) for _ in range(2 * NBUF)]
        + [pltpu.SemaphoreType.DMA for _ in range(2 * NBUF)],
    )
    def gather_k(uid_hbm, iid_hbm, ut_hbm, it_hbm, xt_hbm,
                 uidx_v, iidx_v, xt_v, *stage_and_sems):
        stage = stage_and_sems[:2 * NBUF]
        sems = stage_and_sems[2 * NBUF:]
        wid = lax.axis_index("s") * nc + lax.axis_index("c")
        base = wid * b_per_w
        pltpu.sync_copy(uid_hbm.at[pl.ds(base, b_per_w)], uidx_v)
        pltpu.sync_copy(iid_hbm.at[pl.ds(base, b_per_w)], iidx_v)

        iota16 = lax.iota(jnp.int32, 16)
        xpos_lo = iota16 * b_per_w
        xpos_hi = (iota16 + 16) * b_per_w

        def fire(tbl, col, buf, sem):
            tile0 = pl.multiple_of((col >> 7) << 7, LANES)
            pltpu.async_copy(tbl.at[:, pl.ds(tile0, LANES)], buf, sem)

        def extract(e, lvec, buf, sem, xoff):
            pltpu.make_async_copy(
                ut_hbm.at[:, pl.ds(0, LANES)], buf, sem).wait()
            lo = plsc.load_gather(buf, [iota16, lvec])
            hi = plsc.load_gather(buf, [iota16 + 16, lvec])
            ecast = jnp.full((16,), e, jnp.int32)
            plsc.store_scatter(xt_v, [xoff + xpos_lo + ecast], lo)
            plsc.store_scatter(xt_v, [xoff + xpos_hi + ecast], hi)

        @pl.loop(0, g16)
        def _(g):
            uvec = uidx_v[pl.ds(g * 16, 16)]
            ivec = iidx_v[pl.ds(g * 16, 16)]
            ulane = uvec & (LANES - 1)
            ilane = ivec & (LANES - 1)
            for l in range(16):
                bu = (2 * l) % (2 * NBUF)
                bi = (2 * l + 1) % (2 * NBUF)
                if l >= NBUF:
                    lp = l - NBUF
                    extract(g * 16 + lp,
                            jnp.full((16,), ulane[lp], jnp.int32),
                            stage[bu], sems[bu], 0)
                    extract(g * 16 + lp,
                            jnp.full((16,), ilane[lp], jnp.int32),
                            stage[bi], sems[bi], EMBED * b_per_w)
                fire(ut_hbm, uvec[l], stage[bu], sems[bu])
                fire(it_hbm, ivec[l], stage[bi], sems[bi])
            for l in range(16 - NBUF, 16):
                bu = (2 * l) % (2 * NBUF)
                bi = (2 * l + 1) % (2 * NBUF)
                extract(g * 16 + l, jnp.full((16,), ulane[l], jnp.int32),
                        stage[bu], sems[bu], 0)
                extract(g * 16 + l, jnp.full((16,), ilane[l], jnp.int32),
                        stage[bi], sems[bi], EMBED * b_per_w)

        for c in range(2 * EMBED):
            pltpu.sync_copy(
                xt_v.at[pl.ds(c * b_per_w, b_per_w)],
                xt_hbm.at[pl.ds(c * BATCH + base, b_per_w)])

    return gather_k(user_id, item_id, ut_t, it_t)


def _mlp_body(xt_ref, w1_ref, b1_ref, w2_ref, b2_ref, out_ref):
    h = jnp.dot(w1_ref[...], xt_ref[...], preferred_element_type=jnp.float32)
    h = jnp.maximum(h + b1_ref[...], 0.0)
    out_ref[...] = (
        jnp.dot(w2_ref[...], h, preferred_element_type=jnp.float32)
        + b2_ref[0, 0]
    )


def _tc_mlp(xt, W1, b1_col, W2, b2_2d):
    grid = (BATCH // BLK,)
    return pl.pallas_call(
        _mlp_body,
        grid=grid,
        in_specs=[
            pl.BlockSpec((2 * EMBED, BLK), lambda i: (0, i)),
            pl.BlockSpec((HIDDEN, 2 * EMBED), lambda i: (0, 0)),
            pl.BlockSpec((HIDDEN, 1), lambda i: (0, 0)),
            pl.BlockSpec((1, HIDDEN), lambda i: (0, 0)),
            pl.BlockSpec((1, 1), lambda i: (0, 0)),
        ],
        out_specs=pl.BlockSpec((1, BLK), lambda i: (0, i)),
        out_shape=jax.ShapeDtypeStruct((1, BATCH), jnp.float32),
    )(xt, W1, b1_col, W2, b2_2d)


def kernel(user_id, item_id, user_table, item_table, W1, b1, W2, b2):
    uid = user_id.astype(jnp.int32)
    iid = item_id.astype(jnp.int32)
    xt_flat = _sc_gather(uid, iid, user_table.T, item_table.T)
    xt = xt_flat.reshape(2 * EMBED, BATCH)
    out = _tc_mlp(xt, W1, b1.reshape(HIDDEN, 1), W2, b2.reshape(1, 1))
    return out.reshape(BATCH)
